# baseline (speedup 1.0000x reference)
import functools

import jax
import jax.numpy as jnp
from jax.experimental import pallas as pl

EPS = 1e-12


def _embed_ln_kernel(x_ref, pos_ref, gamma_ref, beta_ref, out_ref):
    x = x_ref[...]
    pos = pos_ref[...]
    e = x + pos[None, :, :]
    mean = jnp.mean(e, axis=-1, keepdims=True)
    c = e - mean
    var = jnp.mean(c * c, axis=-1, keepdims=True)
    normed = c * jax.lax.rsqrt(var + EPS)
    out_ref[...] = normed * gamma_ref[...][None, None, :] + beta_ref[...][None, None, :]


@functools.partial(jax.jit, static_argnames=("blk",))
def _run(methylation_data, pos_table, gamma, beta, blk):
    B, S, H = methylation_data.shape
    grid = (S // blk,)
    return pl.pallas_call(
        _embed_ln_kernel,
        grid=grid,
        in_specs=[
            pl.BlockSpec((B, blk, H), lambda j: (0, j, 0)),
            pl.BlockSpec((blk, H), lambda j: (j, 0)),
            pl.BlockSpec((H,), lambda j: (0,)),
            pl.BlockSpec((H,), lambda j: (0,)),
        ],
        out_specs=pl.BlockSpec((B, blk, H), lambda j: (0, j, 0)),
        out_shape=jax.ShapeDtypeStruct((B, S, H), methylation_data.dtype),
    )(methylation_data, pos_table, gamma, beta)


def kernel(methylation_data, pos_table, gamma, beta):
    S = methylation_data.shape[1]
    return _run(methylation_data, pos_table[:S], gamma, beta, blk=256)


# speedup vs baseline: 1.1095x; 1.1095x over previous
import functools

import jax
import jax.numpy as jnp
from jax.experimental import pallas as pl

EPS = 1e-12


def _embed_ln_kernel(x_ref, pos_ref, gamma_ref, beta_ref, out_ref):
    x = x_ref[...]
    pos = pos_ref[...]
    e = x + pos[None, :, :]
    mean = jnp.mean(e, axis=-1, keepdims=True)
    c = e - mean
    var = jnp.mean(c * c, axis=-1, keepdims=True)
    normed = c * jax.lax.rsqrt(var + EPS)
    out_ref[...] = normed * gamma_ref[...][None, None, :] + beta_ref[...][None, None, :]


@functools.partial(jax.jit, static_argnames=("blk",))
def _run(methylation_data, pos_table, gamma, beta, blk):
    B, S, H = methylation_data.shape
    grid = (S // blk,)
    return pl.pallas_call(
        _embed_ln_kernel,
        grid=grid,
        in_specs=[
            pl.BlockSpec((B, blk, H), lambda j: (0, j, 0)),
            pl.BlockSpec((blk, H), lambda j: (j, 0)),
            pl.BlockSpec((H,), lambda j: (0,)),
            pl.BlockSpec((H,), lambda j: (0,)),
        ],
        out_specs=pl.BlockSpec((B, blk, H), lambda j: (0, j, 0)),
        out_shape=jax.ShapeDtypeStruct((B, S, H), methylation_data.dtype),
    )(methylation_data, pos_table, gamma, beta)


def kernel(methylation_data, pos_table, gamma, beta):
    S = methylation_data.shape[1]
    return _run(methylation_data, pos_table[:S], gamma, beta, blk=768)
